# Initial kernel scaffold; baseline (speedup 1.0000x reference)
import jax
import jax.numpy as jnp
from jax.experimental import pallas as pl


def kernel(x, pos_emb_weight):
    raise NotImplementedError("write your pallas kernel here")


# speedup vs baseline: 1.7219x; 1.7219x over previous
import jax
import jax.numpy as jnp
from jax.experimental import pallas as pl


def _add_kernel(x_ref, pos_ref, out_ref):
    out_ref[...] = x_ref[...] + pos_ref[...][None, :, :]


def kernel(x, pos_emb_weight):
    Bx, Tx, Dx = x.shape
    TB = 256
    grid = (Tx // TB,)
    return pl.pallas_call(
        _add_kernel,
        grid=grid,
        in_specs=[
            pl.BlockSpec((Bx, TB, Dx), lambda t: (0, t, 0)),
            pl.BlockSpec((TB, Dx), lambda t: (t, 0)),
        ],
        out_specs=pl.BlockSpec((Bx, TB, Dx), lambda t: (0, t, 0)),
        out_shape=jax.ShapeDtypeStruct((Bx, Tx, Dx), x.dtype),
    )(x, pos_emb_weight[:Tx])


# speedup vs baseline: 1.7335x; 1.0068x over previous
import jax
import jax.numpy as jnp
from jax.experimental import pallas as pl


def _add_kernel(x_ref, pos_ref, out_ref):
    out_ref[...] = x_ref[...] + pos_ref[...][None, :, :]


def kernel(x, pos_emb_weight):
    Bx, Tx, Dx = x.shape
    TB = 512
    grid = (Tx // TB,)
    return pl.pallas_call(
        _add_kernel,
        grid=grid,
        in_specs=[
            pl.BlockSpec((Bx, TB, Dx), lambda t: (0, t, 0)),
            pl.BlockSpec((TB, Dx), lambda t: (t, 0)),
        ],
        out_specs=pl.BlockSpec((Bx, TB, Dx), lambda t: (0, t, 0)),
        out_shape=jax.ShapeDtypeStruct((Bx, Tx, Dx), x.dtype),
    )(x, pos_emb_weight[:Tx])
